# Initial kernel scaffold; baseline (speedup 1.0000x reference)
import jax
import jax.numpy as jnp
from jax.experimental import pallas as pl


def kernel(query_source, context, to_q_w, to_kv_w, to_out_w, ctx_g, ctx_b, qs_g, qs_b, on_g, on_b, gamma):
    raise NotImplementedError("write your pallas kernel here")


# speedup vs baseline: 12.2870x; 12.2870x over previous
import functools

import jax
import jax.numpy as jnp
from jax import lax
from jax.experimental import pallas as pl
from jax.experimental.pallas import tpu as pltpu

DIM = 384
HEADS = 8
DIM_HEAD = 64
INNER = HEADS * DIM_HEAD
TOP_K = 128
KM_ITERS = 10

F32 = jnp.float32
HI = lax.Precision.HIGHEST


def _ln(x, g, b, eps=1e-5):
    mean = jnp.mean(x, axis=1, keepdims=True)
    var = jnp.mean((x - mean) ** 2, axis=1, keepdims=True)
    return (x - mean) / jnp.sqrt(var + eps) * g + b


def _l2n(x):
    n = jnp.sqrt(jnp.sum(x * x, axis=1, keepdims=True))
    return x / jnp.maximum(n, 1e-12)


def _proj_kernel(qs_ref, ctx_ref, qw_ref, kvw_ref, ctxg_ref, ctxb_ref,
                 qsg_ref, qsb_ref, q_ref, k_ref, v_ref):
    qs_ln = _ln(qs_ref[0], qsg_ref[...], qsb_ref[...])
    ctx_ln = _ln(ctx_ref[0], ctxg_ref[...], ctxb_ref[...])
    for h in range(HEADS):
        sl = slice(h * DIM_HEAD, (h + 1) * DIM_HEAD)
        sl_v = slice(INNER + h * DIM_HEAD, INNER + (h + 1) * DIM_HEAD)
        qh = lax.dot_general(qs_ln, qw_ref[sl, :], (((1,), (1,)), ((), ())),
                             preferred_element_type=F32)
        q_ref[0, h] = _l2n(qh)
        kh = lax.dot_general(ctx_ln, kvw_ref[sl, :], (((1,), (1,)), ((), ())),
                             preferred_element_type=F32)
        k_ref[0, h] = _l2n(kh)
        v_ref[0, h] = lax.dot_general(ctx_ln, kvw_ref[sl_v, :],
                                      (((1,), (1,)), ((), ())),
                                      preferred_element_type=F32)


def _assign_onehot(xb, c, csq_col, L):
    ones = jnp.ones((1, DIM_HEAD), F32)
    xsq = lax.dot_general(ones, xb * xb, (((1,), (1,)), ((), ())),
                          precision=HI, preferred_element_type=F32)
    mm = lax.dot_general(c, xb, (((1,), (1,)), ((), ())),
                         preferred_element_type=F32)
    d = (xsq + csq_col) - 2.0 * mm
    m = jnp.min(d, axis=0, keepdims=True)
    iota = lax.broadcasted_iota(jnp.int32, (TOP_K, L), 0)
    idx = jnp.min(jnp.where(d == m, iota, TOP_K), axis=0, keepdims=True)
    return (iota == idx).astype(F32)


def _csq_col(c):
    return jnp.sum(c * c, axis=1, keepdims=True)


def _kmeans_kernel(q_ref, c_ref, *, n_bh, L):
    c0 = q_ref[0, 0:TOP_K, :]
    ones_l = jnp.ones((L, 1), F32)

    def iter_body(_, c):
        csq = _csq_col(c)
        sums = jnp.zeros((TOP_K, DIM_HEAD), F32)
        counts = jnp.zeros((TOP_K, 1), F32)
        for bh in range(n_bh):
            onehot = _assign_onehot(q_ref[bh], c, csq, L)
            sums = sums + lax.dot_general(
                onehot, q_ref[bh], (((1,), (0,)), ((), ())),
                precision=HI, preferred_element_type=F32)
            counts = counts + lax.dot_general(
                onehot, ones_l, (((1,), (0,)), ((), ())),
                precision=HI, preferred_element_type=F32)
        return jnp.where(counts > 0, sums / jnp.maximum(counts, 1.0), c)

    c_ref[...] = lax.fori_loop(0, KM_ITERS, iter_body, c0)


def _attn_kernel(q_ref, k_ref, v_ref, c_ref, o_ref, *, L):
    qb, kb, vb, c = q_ref[0], k_ref[0], v_ref[0], c_ref[...]
    onehot = _assign_onehot(kb, c, _csq_col(c), L)
    kc = lax.dot_general(onehot, c, (((0,), (0,)), ((), ())),
                         precision=HI, preferred_element_type=F32)
    ad = jnp.abs(kc - kb)
    ones = jnp.ones((1, DIM_HEAD), F32)
    kdist = lax.dot_general(ones, ad, (((1,), (1,)), ((), ())),
                            precision=HI, preferred_element_type=F32)

    def bis(_, carry):
        lo, hi = carry
        mid = (lo + hi) * 0.5
        big = jnp.sum((kdist >= mid).astype(jnp.int32)) >= TOP_K
        return jnp.where(big, mid, lo), jnp.where(big, hi, mid)

    lo, _ = lax.fori_loop(0, 48, bis, (jnp.min(kdist), jnp.max(kdist) + 1.0))
    gt = kdist > lo
    eq = kdist == lo
    need = (TOP_K - jnp.sum(gt.astype(jnp.int32))).astype(F32)
    ii = lax.broadcasted_iota(jnp.int32, (L, L), 0)
    jj = lax.broadcasted_iota(jnp.int32, (L, L), 1)
    tri = (ii <= jj).astype(F32)
    cum = lax.dot_general(eq.astype(F32), tri, (((1,), (0,)), ((), ())),
                          precision=HI, preferred_element_type=F32)
    mask = gt | (eq & (cum <= need))

    sim = lax.dot_general(qb, kb, (((1,), (1,)), ((), ())),
                          preferred_element_type=F32)
    sim = jnp.where(mask, sim, -1e30)
    mx = jnp.max(sim, axis=1, keepdims=True)
    p = jnp.exp(sim - mx)
    attn = p / jnp.sum(p, axis=1, keepdims=True)
    o_ref[0] = lax.dot_general(attn, vb, (((1,), (0,)), ((), ())),
                               preferred_element_type=F32)


def _out_kernel(o_ref, w_ref, g_ref, b_ref, gam_ref, qs_ref, y_ref):
    ot = jnp.concatenate([o_ref[0, h] for h in range(HEADS)], axis=1)
    y = lax.dot_general(ot, w_ref[...], (((1,), (1,)), ((), ())),
                        preferred_element_type=F32)
    y_ref[0] = gam_ref[0, 0] * _ln(y, g_ref[...], b_ref[...]) + qs_ref[0]


def kernel(query_source, context, to_q_w, to_kv_w, to_out_w, ctx_g, ctx_b,
           qs_g, qs_b, on_g, on_b, gamma):
    b, C, H, W = query_source.shape
    L = H * W
    n_bh = b * HEADS
    qs = jnp.transpose(query_source.reshape(b, C, L), (0, 2, 1))
    ctx = jnp.transpose(context.reshape(b, C, L), (0, 2, 1))
    row = lambda t: t.reshape(1, C)

    full = lambda shp: pl.BlockSpec(shp, lambda i: (0,) * len(shp))
    perb = lambda shp: pl.BlockSpec(shp, lambda i: (i,) + (0,) * (len(shp) - 1))

    q, k, v = pl.pallas_call(
        _proj_kernel,
        grid=(b,),
        in_specs=[perb((1, L, C)), perb((1, L, C)), full((INNER, C)),
                  full((2 * INNER, C)), full((1, C)), full((1, C)),
                  full((1, C)), full((1, C))],
        out_specs=[perb((1, HEADS, L, DIM_HEAD))] * 3,
        out_shape=[jax.ShapeDtypeStruct((b, HEADS, L, DIM_HEAD), F32)] * 3,
    )(qs, ctx, to_q_w, to_kv_w, row(ctx_g), row(ctx_b), row(qs_g), row(qs_b))

    qh = q.reshape(n_bh, L, DIM_HEAD)
    kh = k.reshape(n_bh, L, DIM_HEAD)
    vh = v.reshape(n_bh, L, DIM_HEAD)

    cent = pl.pallas_call(
        functools.partial(_kmeans_kernel, n_bh=n_bh, L=L),
        in_specs=[pl.BlockSpec((n_bh, L, DIM_HEAD), lambda: (0, 0, 0))],
        out_specs=pl.BlockSpec((TOP_K, DIM_HEAD), lambda: (0, 0)),
        out_shape=jax.ShapeDtypeStruct((TOP_K, DIM_HEAD), F32),
    )(qh)

    o = pl.pallas_call(
        functools.partial(_attn_kernel, L=L),
        grid=(n_bh,),
        in_specs=[perb((1, L, DIM_HEAD))] * 3 + [full((TOP_K, DIM_HEAD))],
        out_specs=perb((1, L, DIM_HEAD)),
        out_shape=jax.ShapeDtypeStruct((n_bh, L, DIM_HEAD), F32),
    )(qh, kh, vh, cent)

    y = pl.pallas_call(
        _out_kernel,
        grid=(b,),
        in_specs=[perb((1, HEADS, L, DIM_HEAD)), full((DIM, INNER)),
                  full((1, C)), full((1, C)),
                  pl.BlockSpec(memory_space=pltpu.SMEM),
                  perb((1, L, C))],
        out_specs=perb((1, L, C)),
        out_shape=jax.ShapeDtypeStruct((b, L, C), F32),
    )(o.reshape(b, HEADS, L, DIM_HEAD), to_out_w, row(on_g), row(on_b),
      gamma.reshape(1, 1), qs)

    return jnp.transpose(y, (0, 2, 1)).reshape(b, C, H, W)
